# Initial kernel scaffold; baseline (speedup 1.0000x reference)
import jax
import jax.numpy as jnp
from jax.experimental import pallas as pl


def kernel(x, W_in, keys_a, keys_b, expert_u, expert_v, W_out, gamma, beta):
    raise NotImplementedError("write your pallas kernel here")


# speedup vs baseline: 1.0056x; 1.0056x over previous
import functools
import jax
import jax.numpy as jnp
from jax.experimental import pallas as pl
from jax.experimental.pallas import tpu as pltpu

B_, T_, D_ = 2, 2048, 1024
H_, K_, S_ = 8, 8, 512
HD_ = D_ // H_
N_ = B_ * T_


def _out_ln_body(m_ref, w_ref, g_ref, b_ref, o_ref):
    y = jnp.dot(m_ref[...], w_ref[...].T, preferred_element_type=jnp.float32)
    mu = jnp.mean(y, axis=-1, keepdims=True)
    var = jnp.mean((y - mu) ** 2, axis=-1, keepdims=True)
    yn = (y - mu) * jax.lax.rsqrt(var + 1e-5)
    o_ref[...] = yn * g_ref[...] + b_ref[...]


def _out_ln(merged, W_out, gamma, beta):
    BT = 512
    return pl.pallas_call(
        _out_ln_body,
        grid=(N_ // BT,),
        in_specs=[
            pl.BlockSpec((BT, D_), lambda i: (i, 0)),
            pl.BlockSpec((D_, D_), lambda i: (0, 0)),
            pl.BlockSpec((1, D_), lambda i: (0, 0)),
            pl.BlockSpec((1, D_), lambda i: (0, 0)),
        ],
        out_specs=pl.BlockSpec((BT, D_), lambda i: (i, 0)),
        out_shape=jax.ShapeDtypeStruct((N_, D_), jnp.float32),
    )(merged, W_out, gamma.reshape(1, D_), beta.reshape(1, D_))


def kernel(x, W_in, keys_a, keys_b, expert_u, expert_v, W_out, gamma, beta):
    h = (x.reshape(N_, D_) @ W_in.T).reshape(N_, H_, HD_)
    scores_a = jnp.einsum('nhd,hsd->nhs', h, keys_a)
    scores_b = jnp.einsum('nhd,hsd->nhs', h, keys_b)
    top_a_scores, top_a_idx = jax.lax.top_k(scores_a, K_)
    top_b_scores, top_b_idx = jax.lax.top_k(scores_b, K_)
    product_scores = top_a_scores[..., :, None] + top_b_scores[..., None, :]
    product_idx = top_a_idx[..., :, None] * S_ + top_b_idx[..., None, :]
    product_scores_flat = product_scores.reshape(N_, H_, K_ * K_)
    product_idx_flat = product_idx.reshape(N_, H_, K_ * K_)
    top_scores, top_pos = jax.lax.top_k(product_scores_flat, K_)
    top_expert_idx = jnp.take_along_axis(product_idx_flat, top_pos, axis=-1)
    top_weights = jax.nn.softmax(top_scores, axis=-1)
    flat_idx = top_expert_idx.reshape(-1)
    sel_u = jnp.take(expert_u, flat_idx, axis=0).reshape(N_, H_, K_, HD_)
    sel_v = jnp.take(expert_v, flat_idx, axis=0).reshape(N_, H_, K_, HD_)
    h_expanded = h[:, :, None, :]
    activations = jax.nn.sigmoid(jnp.sum(h_expanded * sel_u, axis=-1))
    weighted_acts = (top_weights * activations)[..., None]
    expert_out = jnp.sum(weighted_acts * sel_v, axis=2)
    merged = expert_out.reshape(N_, D_)
    out = _out_ln(merged, W_out, gamma, beta)
    return out.reshape(B_, T_, D_)


# speedup vs baseline: 3.8748x; 3.8534x over previous
import functools
import jax
import jax.numpy as jnp
from jax import lax
from jax.experimental import pallas as pl
from jax.experimental.pallas import tpu as pltpu

B_, T_, D_ = 2, 2048, 1024
H_, K_, S_ = 8, 8, 512
HD_ = D_ // H_
N_ = B_ * T_

_NEG = float('-inf')


def _topk_iter(s, iota, k):
    L = s.shape[1]
    vals, idxs = [], []
    for _ in range(k):
        m = jnp.max(s, axis=1, keepdims=True)
        eq = s == m
        iv = jnp.min(jnp.where(eq, iota, L), axis=1, keepdims=True)
        vals.append(m)
        idxs.append(iv)
        s = jnp.where(iota == iv, _NEG, s)
    return jnp.concatenate(vals, axis=1), jnp.concatenate(idxs, axis=1)


def _router_body(x_ref, w_ref, ka_ref, kb_ref, h_ref, wout_ref, iout_ref):
    BT = x_ref.shape[0]
    h = jnp.dot(x_ref[...], w_ref[...].T, preferred_element_type=jnp.float32)
    h_ref[...] = h
    iota_s = lax.broadcasted_iota(jnp.int32, (1, S_), 1)
    iota_p = lax.broadcasted_iota(jnp.int32, (1, K_ * K_), 1)
    w_parts, i_parts = [], []
    for hd in range(H_):
        hh = h[:, hd * HD_:(hd + 1) * HD_]
        sa = lax.dot_general(hh, ka_ref[hd], (((1,), (1,)), ((), ())),
                             preferred_element_type=jnp.float32)
        sb = lax.dot_general(hh, kb_ref[hd], (((1,), (1,)), ((), ())),
                             preferred_element_type=jnp.float32)
        va, ia = _topk_iter(sa, iota_s, K_)
        vb, ib = _topk_iter(sb, iota_s, K_)
        pv = (va[:, :, None] + vb[:, None, :]).reshape(BT, K_ * K_)
        pi = (ia[:, :, None] * S_ + ib[:, None, :]).reshape(BT, K_ * K_)
        tv, tpos = _topk_iter(pv, iota_p, K_)
        ti = []
        for j in range(K_):
            sel = iota_p == tpos[:, j:j + 1]
            ti.append(jnp.min(jnp.where(sel, pi, jnp.int32(2147483647)),
                              axis=1, keepdims=True))
        ti = jnp.concatenate(ti, axis=1)
        mx = jnp.max(tv, axis=1, keepdims=True)
        e = jnp.exp(tv - mx)
        w = e / jnp.sum(e, axis=1, keepdims=True)
        w_parts.append(w)
        i_parts.append(ti)
    wout_ref[...] = jnp.concatenate(w_parts, axis=1)
    iout_ref[...] = jnp.concatenate(i_parts, axis=1)


def _router(x2d, W_in, keys_a, keys_b):
    BT = 256
    return pl.pallas_call(
        _router_body,
        grid=(N_ // BT,),
        in_specs=[
            pl.BlockSpec((BT, D_), lambda i: (i, 0)),
            pl.BlockSpec((D_, D_), lambda i: (0, 0)),
            pl.BlockSpec((H_, S_, HD_), lambda i: (0, 0, 0)),
            pl.BlockSpec((H_, S_, HD_), lambda i: (0, 0, 0)),
        ],
        out_specs=[
            pl.BlockSpec((BT, D_), lambda i: (i, 0)),
            pl.BlockSpec((BT, H_ * K_), lambda i: (i, 0)),
            pl.BlockSpec((BT, H_ * K_), lambda i: (i, 0)),
        ],
        out_shape=[
            jax.ShapeDtypeStruct((N_, D_), jnp.float32),
            jax.ShapeDtypeStruct((N_, H_ * K_), jnp.float32),
            jax.ShapeDtypeStruct((N_, H_ * K_), jnp.int32),
        ],
    )(x2d, W_in, keys_a, keys_b)


def _out_ln_body(m_ref, w_ref, g_ref, b_ref, o_ref):
    y = jnp.dot(m_ref[...], w_ref[...].T, preferred_element_type=jnp.float32)
    mu = jnp.mean(y, axis=-1, keepdims=True)
    var = jnp.mean((y - mu) ** 2, axis=-1, keepdims=True)
    yn = (y - mu) * lax.rsqrt(var + 1e-5)
    o_ref[...] = yn * g_ref[...] + b_ref[...]


def _out_ln(merged, W_out, gamma, beta):
    BT = 512
    return pl.pallas_call(
        _out_ln_body,
        grid=(N_ // BT,),
        in_specs=[
            pl.BlockSpec((BT, D_), lambda i: (i, 0)),
            pl.BlockSpec((D_, D_), lambda i: (0, 0)),
            pl.BlockSpec((1, D_), lambda i: (0, 0)),
            pl.BlockSpec((1, D_), lambda i: (0, 0)),
        ],
        out_specs=pl.BlockSpec((BT, D_), lambda i: (i, 0)),
        out_shape=jax.ShapeDtypeStruct((N_, D_), jnp.float32),
    )(merged, W_out, gamma.reshape(1, D_), beta.reshape(1, D_))


def kernel(x, W_in, keys_a, keys_b, expert_u, expert_v, W_out, gamma, beta):
    h2d, w_flat, i_flat = _router(x.reshape(N_, D_), W_in, keys_a, keys_b)
    h = h2d.reshape(N_, H_, HD_)
    top_weights = w_flat.reshape(N_, H_, K_)
    flat_idx = i_flat.reshape(-1)
    sel_u = jnp.take(expert_u, flat_idx, axis=0).reshape(N_, H_, K_, HD_)
    sel_v = jnp.take(expert_v, flat_idx, axis=0).reshape(N_, H_, K_, HD_)
    h_expanded = h[:, :, None, :]
    activations = jax.nn.sigmoid(jnp.sum(h_expanded * sel_u, axis=-1))
    weighted_acts = (top_weights * activations)[..., None]
    expert_out = jnp.sum(weighted_acts * sel_v, axis=2)
    merged = expert_out.reshape(N_, D_)
    out = _out_ln(merged, W_out, gamma, beta)
    return out.reshape(B_, T_, D_)
